# baseline (speedup 1.0000x reference)
import functools

import jax
import jax.numpy as jnp
from jax import lax
from jax.experimental import pallas as pl
from jax.experimental.pallas import tpu as pltpu
from jax.experimental.pallas import tpu_sc as plsc

B = 16384
D = 64
N_ROWS = 100000
NC = 2
NS = 16
NW = NC * NS
JOBS = 12
H0 = 53248
H1 = N_ROWS - H0
BQ = 4096
NQ = B // BQ


def _sc_body(users, items, t_eu, t_mu, t_fu, t_ei, t_mi, t_fi,
             o_eu, o_mu, o_fu, o_ei, o_mi, o_fi,
             idx_vm, h0, h1, ow, sem_h0, sem_h1, sem_out):
    wid = lax.axis_index("s") * NC + lax.axis_index("c")

    def side_body(w, idx_src, tabs, outs):
        pltpu.sync_copy(idx_src, idx_vm)

        def dim_of(j):
            return w * 4 + (j % 4)

        def wait_h0(tab, d):
            pltpu.make_async_copy(tab.at[d, pl.ds(0, H0)], h0, sem_h0).wait()

        def wait_h1(tab, d):
            pltpu.make_async_copy(tab.at[d, pl.ds(H0, H1)], h1, sem_h1).wait()

        def drain_wb(out, d):
            pltpu.make_async_copy(ow.at[pl.ds(0, BQ)],
                                  out.at[d, pl.ds(0, BQ)], sem_out).wait()

        pltpu.async_copy(tabs[0].at[dim_of(0), pl.ds(0, H0)], h0, sem_h0)
        pltpu.async_copy(tabs[0].at[dim_of(0), pl.ds(H0, H1)], h1, sem_h1)
        for j in range(JOBS):
            tab = tabs[j // 4]
            out = outs[j // 4]
            d = dim_of(j)
            jn = j + 1
            wait_h0(tab, d)

            def quarter(q, carry):
                obase = lax.rem(q, 2) * BQ

                def do_drain():
                    drain_wb(out, d)

                if j == 0:
                    pl.when(q >= 2)(do_drain)
                else:
                    do_drain()

                def p0(k, c2):
                    iv = idx_vm[pl.ds(q * BQ + k * 16, 16)]
                    ivc = jnp.minimum(iv, H0 - 1)
                    ow[pl.ds(obase + k * 16, 16)] = plsc.load_gather(
                        h0, [ivc])
                    return c2

                lax.fori_loop(0, BQ // 16, p0, 0, unroll=8)
                if jn < JOBS:
                    @pl.when(q == NQ - 1)
                    def _():
                        pltpu.async_copy(
                            tabs[jn // 4].at[dim_of(jn), pl.ds(0, H0)], h0,
                            sem_h0)

                @pl.when(q == 0)
                def _():
                    wait_h1(tab, d)

                def p1(k, c2):
                    iv = idx_vm[pl.ds(q * BQ + k * 16, 16)]
                    tc = jnp.maximum(iv - H0, 0)
                    g = plsc.load_gather(h1, [tc])
                    sl = pl.ds(obase + k * 16, 16)
                    ow[sl] = jnp.where(iv >= H0, g, ow[sl])
                    return c2

                lax.fori_loop(0, BQ // 16, p1, 0, unroll=8)
                pltpu.async_copy(ow.at[pl.ds(obase, BQ)],
                                 out.at[d, pl.ds(q * BQ, BQ)], sem_out)
                return carry

            lax.fori_loop(0, NQ, quarter, 0)
            if jn < JOBS:
                pltpu.async_copy(
                    tabs[jn // 4].at[dim_of(jn), pl.ds(H0, H1)], h1, sem_h1)
        drain_wb(outs[-1], dim_of(JOBS - 1))
        drain_wb(outs[-1], dim_of(JOBS - 1))

    @pl.when(wid < 16)
    def _():
        side_body(wid, users, (t_eu, t_mu, t_fu), (o_eu, o_mu, o_fu))

    @pl.when(wid >= 16)
    def _():
        side_body(wid - 16, items, (t_ei, t_mi, t_fi), (o_ei, o_mi, o_fi))


@functools.cache
def _sc_gather():
    return pl.kernel(
        _sc_body,
        out_type=[jax.ShapeDtypeStruct((D, B), jnp.float32)] * 6,
        mesh=plsc.VectorSubcoreMesh(core_axis_name="c", subcore_axis_name="s"),
        scratch_types=[
            pltpu.VMEM((B,), jnp.int32),
            pltpu.VMEM((H0,), jnp.float32),
            pltpu.VMEM((H1,), jnp.float32),
            pltpu.VMEM((2 * BQ,), jnp.float32),
            pltpu.SemaphoreType.DMA,
            pltpu.SemaphoreType.DMA,
            pltpu.SemaphoreType.DMA,
        ],
        compiler_params=pltpu.CompilerParams(use_tc_tiling_on_sc=True,
                                             needs_layout_passes=False),
    )


BT = 4096


def _tc_body(eu, mu, fu, ei, mi, fi, w1aT, w1bT, b1, w2a, w2b, b2, out):
    def imask(m):
        return lax.bitcast_convert_type(m[...], jnp.int32).astype(jnp.float32)

    xu = eu[...] * imask(mu)
    xi = ei[...] * imask(mi)
    mf = fu[...] * fi[...]
    h = jnp.dot(w1aT[...], xu, preferred_element_type=jnp.float32)
    h = h + jnp.dot(w1bT[...], xi, preferred_element_type=jnp.float32)
    h = jnp.maximum(h + b1[...], 0.0)
    o = jnp.dot(w2a[...], h, preferred_element_type=jnp.float32)
    o = o + jnp.dot(w2b[...], mf, preferred_element_type=jnp.float32)
    out[...] = o + b2[0, 0]


_tc_call = pl.pallas_call(
    _tc_body,
    grid=(B // BT,),
    in_specs=[pl.BlockSpec((D, BT), lambda n: (0, n))] * 6 + [
        pl.BlockSpec((D, D), lambda n: (0, 0)),
        pl.BlockSpec((D, D), lambda n: (0, 0)),
        pl.BlockSpec((D, 1), lambda n: (0, 0)),
        pl.BlockSpec((1, D), lambda n: (0, 0)),
        pl.BlockSpec((1, D), lambda n: (0, 0)),
        pl.BlockSpec((1, 1), lambda n: (0, 0)),
    ],
    out_specs=pl.BlockSpec((1, BT), lambda n: (0, n)),
    out_shape=jax.ShapeDtypeStruct((1, B), jnp.float32),
)


def kernel(users, items, emb_user_mlp, emb_item_mlp, emb_user_mf, emb_item_mf,
           user_mask, item_mask, W1, b1, W2, b2):
    fbits = lambda m: lax.bitcast_convert_type(m, jnp.float32)
    eu, mu, fu, ei, mi, fi = _sc_gather()(
        users, items,
        emb_user_mlp.T, fbits(user_mask).T, emb_user_mf.T,
        emb_item_mlp.T, fbits(item_mask).T, emb_item_mf.T)
    o = _tc_call(eu, mu, fu, ei, mi, fi,
                 W1[:D].T, W1[D:].T, b1.reshape(D, 1),
                 W2[:D].reshape(1, D), W2[D:].reshape(1, D),
                 b2.reshape(1, 1))
    return o.reshape(B, 1)


# speedup vs baseline: 1.5399x; 1.5399x over previous
import functools

import jax
import jax.numpy as jnp
from jax import lax
from jax.experimental import pallas as pl
from jax.experimental.pallas import tpu as pltpu
from jax.experimental.pallas import tpu_sc as plsc

B = 16384
D = 64
N_ROWS = 100000
NC = 2
NS = 16
NW = NC * NS
JOBS = 12
OC = 1024
NCH = B // OC
OSUB = 8


def _sc_body(users, items, t_eu, t_mu, t_fu, t_ei, t_mi, t_fi,
             o_eu, o_mu, o_fu, o_ei, o_mi, o_fi,
             idx_vm, row_vm, out_vm, sem_out):
    wid = lax.axis_index("s") * NC + lax.axis_index("c")

    def side_body(w, idx_src, tabs, outs):
        pltpu.sync_copy(idx_src, idx_vm)
        for j in range(JOBS):
            tab = tabs[j // 4]
            out = outs[j // 4]
            d = w * 4 + (j % 4)
            pltpu.sync_copy(tab.at[d], row_vm)

            def chunk(c, carry):
                sub = lax.rem(c, OSUB)

                @pl.when(c >= OSUB)
                def _():
                    pltpu.make_async_copy(
                        out_vm.at[0], out.at[d, pl.ds(0, OC)], sem_out).wait()

                def gat(k, carry2):
                    iv = idx_vm[pl.ds((c * (OC // 16) + k) * 16, 16)]
                    out_vm[sub, pl.ds(k * 16, 16)] = plsc.load_gather(
                        row_vm, [iv])
                    return carry2

                lax.fori_loop(0, OC // 16, gat, 0, unroll=8)
                pltpu.async_copy(out_vm.at[sub],
                                 out.at[d, pl.ds(c * OC, OC)], sem_out)
                return carry

            lax.fori_loop(0, NCH, chunk, 0)
            for _ in range(OSUB):
                pltpu.make_async_copy(
                    out_vm.at[0], out.at[d, pl.ds(0, OC)], sem_out).wait()

    @pl.when(wid < 16)
    def _():
        side_body(wid, users, (t_eu, t_mu, t_fu), (o_eu, o_mu, o_fu))

    @pl.when(wid >= 16)
    def _():
        side_body(wid - 16, items, (t_ei, t_mi, t_fi), (o_ei, o_mi, o_fi))


@functools.cache
def _sc_gather():
    return pl.kernel(
        _sc_body,
        out_type=[jax.ShapeDtypeStruct((D, B), jnp.float32)] * 6,
        mesh=plsc.VectorSubcoreMesh(core_axis_name="c", subcore_axis_name="s"),
        scratch_types=[
            pltpu.VMEM((B,), jnp.int32),
            pltpu.VMEM((N_ROWS,), jnp.float32),
            pltpu.VMEM((OSUB, OC), jnp.float32),
            pltpu.SemaphoreType.DMA,
        ],
        compiler_params=pltpu.CompilerParams(use_tc_tiling_on_sc=True,
                                             needs_layout_passes=False),
    )


BT = 4096


def _tc_body(eu, mu, fu, ei, mi, fi, w1aT, w1bT, b1, w2a, w2b, b2, out):
    def imask(m):
        return lax.bitcast_convert_type(m[...], jnp.int32).astype(jnp.float32)

    xu = eu[...] * imask(mu)
    xi = ei[...] * imask(mi)
    mf = fu[...] * fi[...]
    h = jnp.dot(w1aT[...], xu, preferred_element_type=jnp.float32)
    h = h + jnp.dot(w1bT[...], xi, preferred_element_type=jnp.float32)
    h = jnp.maximum(h + b1[...], 0.0)
    o = jnp.dot(w2a[...], h, preferred_element_type=jnp.float32)
    o = o + jnp.dot(w2b[...], mf, preferred_element_type=jnp.float32)
    out[...] = o + b2[0, 0]


_tc_call = pl.pallas_call(
    _tc_body,
    grid=(B // BT,),
    in_specs=[pl.BlockSpec((D, BT), lambda n: (0, n))] * 6 + [
        pl.BlockSpec((D, D), lambda n: (0, 0)),
        pl.BlockSpec((D, D), lambda n: (0, 0)),
        pl.BlockSpec((D, 1), lambda n: (0, 0)),
        pl.BlockSpec((1, D), lambda n: (0, 0)),
        pl.BlockSpec((1, D), lambda n: (0, 0)),
        pl.BlockSpec((1, 1), lambda n: (0, 0)),
    ],
    out_specs=pl.BlockSpec((1, BT), lambda n: (0, n)),
    out_shape=jax.ShapeDtypeStruct((1, B), jnp.float32),
)


def kernel(users, items, emb_user_mlp, emb_item_mlp, emb_user_mf, emb_item_mf,
           user_mask, item_mask, W1, b1, W2, b2):
    fbits = lambda m: lax.bitcast_convert_type(m, jnp.float32)
    eu, mu, fu, ei, mi, fi = _sc_gather()(
        users, items,
        emb_user_mlp.T, fbits(user_mask).T, emb_user_mf.T,
        emb_item_mlp.T, fbits(item_mask).T, emb_item_mf.T)
    o = _tc_call(eu, mu, fu, ei, mi, fi,
                 W1[:D].T, W1[D:].T, b1.reshape(D, 1),
                 W2[:D].reshape(1, D), W2[D:].reshape(1, D),
                 b2.reshape(1, 1))
    return o.reshape(B, 1)


# speedup vs baseline: 1.5458x; 1.0038x over previous
import functools

import jax
import jax.numpy as jnp
from jax import lax
from jax.experimental import pallas as pl
from jax.experimental.pallas import tpu as pltpu
from jax.experimental.pallas import tpu_sc as plsc

B = 16384
D = 64
N_ROWS = 100000
NC = 2
NS = 16
NW = NC * NS
JOBS = 12
OC = 1024
NCH = B // OC
OSUB = 8


def _sc_body(users, items, t_eu, t_mu, t_fu, t_ei, t_mi, t_fi,
             o_eu, o_mu, o_fu, o_ei, o_mi, o_fi,
             idx_vm, row_vm, out_vm, sem_out, sem_row):
    wid = lax.axis_index("s") * NC + lax.axis_index("c")

    def side_body(w, idx_src, tabs, outs):
        rcp = pltpu.async_copy(tabs[0].at[w * 4], row_vm, sem_row)
        pltpu.sync_copy(idx_src, idx_vm)
        for j in range(JOBS):
            out = outs[j // 4]
            d = w * 4 + (j % 4)
            rcp.wait()

            def chunk(c, carry):
                sub = lax.rem(c, OSUB)

                @pl.when(c >= OSUB)
                def _():
                    pltpu.make_async_copy(
                        out_vm.at[0], out.at[d, pl.ds(0, OC)], sem_out).wait()

                def gat(k, carry2):
                    iv = idx_vm[pl.ds((c * (OC // 16) + k) * 16, 16)]
                    out_vm[sub, pl.ds(k * 16, 16)] = plsc.load_gather(
                        row_vm, [iv])
                    return carry2

                lax.fori_loop(0, OC // 16, gat, 0, unroll=8)
                pltpu.async_copy(out_vm.at[sub],
                                 out.at[d, pl.ds(c * OC, OC)], sem_out)
                return carry

            lax.fori_loop(0, NCH, chunk, 0)
            if j + 1 < JOBS:
                rcp = pltpu.async_copy(
                    tabs[(j + 1) // 4].at[w * 4 + ((j + 1) % 4)], row_vm,
                    sem_row)
            for _ in range(OSUB):
                pltpu.make_async_copy(
                    out_vm.at[0], out.at[d, pl.ds(0, OC)], sem_out).wait()

    @pl.when(wid < 16)
    def _():
        side_body(wid, users, (t_eu, t_mu, t_fu), (o_eu, o_mu, o_fu))

    @pl.when(wid >= 16)
    def _():
        side_body(wid - 16, items, (t_ei, t_mi, t_fi), (o_ei, o_mi, o_fi))


@functools.cache
def _sc_gather():
    return pl.kernel(
        _sc_body,
        out_type=[jax.ShapeDtypeStruct((D, B), jnp.float32)] * 6,
        mesh=plsc.VectorSubcoreMesh(core_axis_name="c", subcore_axis_name="s"),
        scratch_types=[
            pltpu.VMEM((B,), jnp.int32),
            pltpu.VMEM((N_ROWS,), jnp.float32),
            pltpu.VMEM((OSUB, OC), jnp.float32),
            pltpu.SemaphoreType.DMA,
            pltpu.SemaphoreType.DMA,
        ],
        compiler_params=pltpu.CompilerParams(use_tc_tiling_on_sc=True,
                                             needs_layout_passes=False),
    )


BT = 4096


def _tc_body(eu, mu, fu, ei, mi, fi, w1aT, w1bT, b1, w2a, w2b, b2, out):
    def imask(m):
        return lax.bitcast_convert_type(m[...], jnp.int32).astype(jnp.float32)

    xu = eu[...] * imask(mu)
    xi = ei[...] * imask(mi)
    mf = fu[...] * fi[...]
    h = jnp.dot(w1aT[...], xu, preferred_element_type=jnp.float32)
    h = h + jnp.dot(w1bT[...], xi, preferred_element_type=jnp.float32)
    h = jnp.maximum(h + b1[...], 0.0)
    o = jnp.dot(w2a[...], h, preferred_element_type=jnp.float32)
    o = o + jnp.dot(w2b[...], mf, preferred_element_type=jnp.float32)
    out[...] = o + b2[0, 0]


_tc_call = pl.pallas_call(
    _tc_body,
    grid=(B // BT,),
    in_specs=[pl.BlockSpec((D, BT), lambda n: (0, n))] * 6 + [
        pl.BlockSpec((D, D), lambda n: (0, 0)),
        pl.BlockSpec((D, D), lambda n: (0, 0)),
        pl.BlockSpec((D, 1), lambda n: (0, 0)),
        pl.BlockSpec((1, D), lambda n: (0, 0)),
        pl.BlockSpec((1, D), lambda n: (0, 0)),
        pl.BlockSpec((1, 1), lambda n: (0, 0)),
    ],
    out_specs=pl.BlockSpec((1, BT), lambda n: (0, n)),
    out_shape=jax.ShapeDtypeStruct((1, B), jnp.float32),
)


def kernel(users, items, emb_user_mlp, emb_item_mlp, emb_user_mf, emb_item_mf,
           user_mask, item_mask, W1, b1, W2, b2):
    fbits = lambda m: lax.bitcast_convert_type(m, jnp.float32)
    eu, mu, fu, ei, mi, fi = _sc_gather()(
        users, items,
        emb_user_mlp.T, fbits(user_mask).T, emb_user_mf.T,
        emb_item_mlp.T, fbits(item_mask).T, emb_item_mf.T)
    o = _tc_call(eu, mu, fu, ei, mi, fi,
                 W1[:D].T, W1[D:].T, b1.reshape(D, 1),
                 W2[:D].reshape(1, D), W2[D:].reshape(1, D),
                 b2.reshape(1, 1))
    return o.reshape(B, 1)
